# baseline (speedup 1.0000x reference)
import jax
import jax.numpy as jnp
from jax.experimental import pallas as pl
from jax.experimental.pallas import tpu as pltpu


def _adapter_body(x_ref, wd_ref, bd_ref, wu_ref, bu_ref, scale_ref, o_ref):
    x = x_ref[...].astype(jnp.bfloat16)
    down = jnp.dot(x, wd_ref[...], preferred_element_type=jnp.float32)
    down = jnp.maximum(down + bd_ref[...], 0.0)
    up = jnp.dot(down.astype(jnp.bfloat16), wu_ref[...],
                 preferred_element_type=jnp.float32)
    o_ref[...] = (up + bu_ref[...]) * scale_ref[0]


def kernel(x, wd_t, bd, wu_t, bu, scale):
    B, S, D = x.shape
    Rp = wd_t.shape[1]
    M = B * S
    x2 = x.reshape(M, D)

    wd_bf = wd_t.astype(jnp.bfloat16)
    wu_bf = wu_t.astype(jnp.bfloat16)
    bd_f = bd.astype(jnp.float32)
    bu_f = bu.astype(jnp.float32)
    sc = scale.astype(jnp.float32).reshape(1)

    TM = 512
    if M % TM != 0:
        TM = 256 if M % 256 == 0 else 8
    grid = (M // TM,)

    def resident(shape):
        return pl.BlockSpec(shape, lambda i: (0, 0))

    out2 = pl.pallas_call(
        _adapter_body,
        out_shape=jax.ShapeDtypeStruct((M, D), x.dtype),
        grid=grid,
        in_specs=[
            pl.BlockSpec((TM, D), lambda i: (i, 0)),
            resident((D, Rp)),
            resident((1, Rp)),
            resident((Rp, D)),
            resident((1, D)),
            pl.BlockSpec(memory_space=pltpu.MemorySpace.SMEM),
        ],
        out_specs=pl.BlockSpec((TM, D), lambda i: (i, 0)),
        compiler_params=pltpu.CompilerParams(
            dimension_semantics=("parallel",),
            vmem_limit_bytes=48 * 1024 * 1024),
    )(x2, wd_bf, bd_f, wu_bf, bu_f, sc)

    return out2.reshape(B, S, D)


# speedup vs baseline: 1.1750x; 1.1750x over previous
import jax
import jax.numpy as jnp
from jax.experimental import pallas as pl
from jax.experimental.pallas import tpu as pltpu


def _adapter_body(x_ref, wd_ref, bd_ref, wu_ref, bu_ref, scale_ref, o_ref):
    x = x_ref[...].astype(jnp.bfloat16)
    down = jnp.dot(x, wd_ref[...], preferred_element_type=jnp.float32)
    down = jnp.maximum(down + bd_ref[...], 0.0)
    up = jnp.dot(down.astype(jnp.bfloat16), wu_ref[...],
                 preferred_element_type=jnp.float32)
    o_ref[...] = (up + bu_ref[...]) * scale_ref[0]


def kernel(x, wd_t, bd, wu_t, bu, scale):
    B, S, D = x.shape
    Rp = wd_t.shape[1]
    M = B * S
    x2 = x.reshape(M, D)

    wd_bf = wd_t.astype(jnp.bfloat16)
    wu_bf = wu_t.astype(jnp.bfloat16)
    bd_f = bd.astype(jnp.float32)
    bu_f = bu.astype(jnp.float32)
    sc = scale.astype(jnp.float32).reshape(1)

    TM = 1024
    if M % TM != 0:
        TM = 256 if M % 256 == 0 else 8
    grid = (M // TM,)

    def resident(shape):
        return pl.BlockSpec(shape, lambda i: (0, 0))

    out2 = pl.pallas_call(
        _adapter_body,
        out_shape=jax.ShapeDtypeStruct((M, D), x.dtype),
        grid=grid,
        in_specs=[
            pl.BlockSpec((TM, D), lambda i: (i, 0)),
            resident((D, Rp)),
            resident((1, Rp)),
            resident((Rp, D)),
            resident((1, D)),
            pl.BlockSpec(memory_space=pltpu.MemorySpace.SMEM),
        ],
        out_specs=pl.BlockSpec((TM, D), lambda i: (i, 0)),
        compiler_params=pltpu.CompilerParams(
            dimension_semantics=("parallel",),
            vmem_limit_bytes=48 * 1024 * 1024),
    )(x2, wd_bf, bd_f, wu_bf, bu_f, sc)

    return out2.reshape(B, S, D)


# speedup vs baseline: 1.2518x; 1.0654x over previous
import jax
import jax.numpy as jnp
from jax.experimental import pallas as pl
from jax.experimental.pallas import tpu as pltpu


def _adapter_body(x_ref, wd_ref, bd_ref, wu_ref, bu_ref, scale_ref, o_ref):
    x = x_ref[...].astype(jnp.bfloat16)
    down = jnp.dot(x, wd_ref[...], preferred_element_type=jnp.float32)
    down = jnp.maximum(down + bd_ref[...], 0.0)
    up = jnp.dot(down.astype(jnp.bfloat16), wu_ref[...],
                 preferred_element_type=jnp.float32)
    o_ref[...] = (up + bu_ref[...]) * scale_ref[0]


def kernel(x, wd_t, bd, wu_t, bu, scale):
    B, S, D = x.shape
    Rp = wd_t.shape[1]
    M = B * S
    x2 = x.reshape(M, D)

    wd_bf = wd_t.astype(jnp.bfloat16)
    wu_bf = wu_t.astype(jnp.bfloat16)
    bd_f = bd.astype(jnp.float32)
    bu_f = bu.astype(jnp.float32)
    sc = scale.astype(jnp.float32).reshape(1)

    TM = 2048
    while TM > 8 and M % TM != 0:
        TM //= 2
    steps = M // TM
    grid = (steps,)
    semantics = ("parallel",)
    x_map = lambda i: (i, 0)
    w_map = lambda i: (0, 0)

    def resident(shape):
        return pl.BlockSpec(shape, w_map)

    out2 = pl.pallas_call(
        _adapter_body,
        out_shape=jax.ShapeDtypeStruct((M, D), x.dtype),
        grid=grid,
        in_specs=[
            pl.BlockSpec((TM, D), x_map),
            resident((D, Rp)),
            resident((1, Rp)),
            resident((Rp, D)),
            resident((1, D)),
            pl.BlockSpec(memory_space=pltpu.MemorySpace.SMEM),
        ],
        out_specs=pl.BlockSpec((TM, D), x_map),
        compiler_params=pltpu.CompilerParams(
            dimension_semantics=semantics,
            vmem_limit_bytes=48 * 1024 * 1024),
    )(x2, wd_bf, bd_f, wu_bf, bu_f, sc)

    return out2.reshape(B, S, D)


# speedup vs baseline: 1.2572x; 1.0043x over previous
import jax
import jax.numpy as jnp
from jax.experimental import pallas as pl
from jax.experimental.pallas import tpu as pltpu


def _adapter_body(x_ref, wd_ref, bd_ref, wu_ref, bu_ref, o_ref):
    x = x_ref[...].astype(jnp.bfloat16)
    down = jnp.dot(x, wd_ref[...], preferred_element_type=jnp.float32)
    down = jnp.maximum(down + bd_ref[...], 0.0)
    up = jnp.dot(down.astype(jnp.bfloat16), wu_ref[...],
                 preferred_element_type=jnp.float32)
    o_ref[...] = up + bu_ref[...]


def kernel(x, wd_t, bd, wu_t, bu, scale):
    B, S, D = x.shape
    Rp = wd_t.shape[1]
    M = B * S
    x2 = x.reshape(M, D)

    sc = scale.astype(jnp.float32).reshape(())
    wd_bf = wd_t.astype(jnp.bfloat16)
    wu_bf = (wu_t.astype(jnp.float32) * sc).astype(jnp.bfloat16)
    bd_f = bd.astype(jnp.float32)
    bu_f = bu.astype(jnp.float32) * sc

    TM = 2048
    while TM > 8 and M % TM != 0:
        TM //= 2
    steps = M // TM
    grid = (steps,)
    semantics = ("parallel",)
    x_map = lambda i: (i, 0)
    w_map = lambda i: (0, 0)

    def resident(shape):
        return pl.BlockSpec(shape, w_map)

    out2 = pl.pallas_call(
        _adapter_body,
        out_shape=jax.ShapeDtypeStruct((M, D), x.dtype),
        grid=grid,
        in_specs=[
            pl.BlockSpec((TM, D), x_map),
            resident((D, Rp)),
            resident((1, Rp)),
            resident((Rp, D)),
            resident((1, D)),
        ],
        out_specs=pl.BlockSpec((TM, D), x_map),
        compiler_params=pltpu.CompilerParams(
            dimension_semantics=semantics,
            vmem_limit_bytes=48 * 1024 * 1024),
    )(x2, wd_bf, bd_f, wu_bf, bu_f)

    return out2.reshape(B, S, D)
